# baseline (speedup 1.0000x reference)
import jax
import jax.numpy as jnp
from jax import lax
from jax.experimental import pallas as pl
from jax.experimental.pallas import tpu as pltpu
from jax.experimental.pallas import tpu_sc as plsc

_B, _N, _C = 16, 65536, 10
_BN = _B * _N
_MIN_ROI = (0.0, -51.2, -5.0)
_GRID = 0.4

_NC, _NS, _L = 2, 16, 16
_NW = _NC * _NS
_NG = _BN // 128
_GW = _NG // _NW
_GC = 32
_NCH = _GW // _GC
_NBUF = 2


def _ceil_idx(v, min_v):
    q = (v - jnp.float32(min_v)) / jnp.float32(_GRID)
    t = q.astype(jnp.int32)
    tf = t.astype(jnp.float32)
    return jnp.where(q > tf, t + 1, t)


def _sc_body(in_hbm, idx_hbm, in_v, out_v, sem_in, sem_idx):
    wid = lax.axis_index("s") * _NC + lax.axis_index("c")
    b = wid // 2
    g0 = wid * _GW
    nh0 = (wid % 2) * _GW
    b_hi = b // 8
    b_lo = b % 8
    x_min, y_min, z_min = _MIN_ROI

    vb = jnp.full((_L,), b, dtype=jnp.int32)

    def prefill(i, cy):
        for u in range(_NBUF):
            for k in range(128 // _L):
                out_v[u, i, 0, pl.ds(k * _L, _L)] = vb
        return cy

    lax.fori_loop(0, _GC, prefill, 0)

    def issue_in(ch, buf):
        tc = nh0 + ch * _GC
        return [
            pltpu.async_copy(
                in_hbm.at[c, b_hi, pl.ds(tc, _GC), b_lo, :],
                in_v.at[buf, c], sem_in)
            for c in range(3)
        ]

    in_h = [None] * _NCH
    idx_h = [None] * _NCH
    for ch in range(_NBUF):
        in_h[ch] = issue_in(ch, ch % _NBUF)

    for ch in range(_NCH):
        buf = ch % _NBUF
        for h in in_h[ch]:
            h.wait()
        if ch >= _NBUF:
            idx_h[ch - _NBUF].wait()

        def step(i, cy, buf=buf):
            for k in range(128 // _L):
                s = pl.ds(k * _L, _L)
                x = in_v[buf, 0, i, s]
                y = in_v[buf, 1, i, s]
                z = in_v[buf, 2, i, s]
                out_v[buf, i, 1, s] = _ceil_idx(z, z_min)
                out_v[buf, i, 2, s] = _ceil_idx(y, y_min)
                out_v[buf, i, 3, s] = _ceil_idx(x, x_min)
            return cy

        lax.fori_loop(0, _GC, step, 0)
        g = g0 + ch * _GC
        idx_h[ch] = pltpu.async_copy(
            out_v.at[buf], idx_hbm.at[pl.ds(g, _GC), :, :], sem_idx)
        if ch + _NBUF < _NCH:
            in_h[ch + _NBUF] = issue_in(ch + _NBUF, buf)

    for ch in range(_NCH - _NBUF, _NCH):
        idx_h[ch].wait()


_sc_call = pl.kernel(
    _sc_body,
    out_type=jax.ShapeDtypeStruct((_NG, 4, 128), jnp.int32),
    mesh=plsc.VectorSubcoreMesh(core_axis_name="c", subcore_axis_name="s"),
    compiler_params=pltpu.CompilerParams(needs_layout_passes=False),
    scratch_types=[
        pltpu.VMEM((_NBUF, 3, _GC, 128), jnp.float32),
        pltpu.VMEM((_NBUF, _GC, 4, 128), jnp.int32),
        pltpu.SemaphoreType.DMA,
        pltpu.SemaphoreType.DMA,
    ],
)

_NHC = 64


def _tc_body(x_ref, o_ref):
    for c in range(_C):
        for bl in range(8):
            o_ref[c // 8, 0, bl, :, c % 8, :] = x_ref[c, 0, :, bl, :]


_tc_call = pl.pallas_call(
    _tc_body,
    grid=(2, 512 // _NHC),
    in_specs=[pl.BlockSpec(
        (_C, 1, _NHC, 8, 128),
        lambda bh, nc: (0, bh, nc, 0, 0))],
    out_specs=pl.BlockSpec(
        (2, 1, 8, _NHC, 8, 128),
        lambda bh, nc: (0, bh, 0, nc, 0, 0)),
    out_shape=jax.ShapeDtypeStruct((2, 2, 8, 512, 8, 128), jnp.float32),
)


def kernel(rdr_sparse_cube):
    xv = (rdr_sparse_cube.transpose(2, 0, 1)
          .reshape(_C, 2, 8, 512, 128)
          .transpose(0, 1, 3, 2, 4))
    i3 = _sc_call(xv)
    f6 = _tc_call(xv)
    feat = (f6.reshape(2, _NG, 8, 128)
            .transpose(0, 2, 1, 3).reshape(16, _BN).T)[:, :_C]
    idx = i3.transpose(0, 2, 1).reshape(_BN, 4)
    return feat, idx


# speedup vs baseline: 1.0672x; 1.0672x over previous
import jax
import jax.numpy as jnp
from jax import lax
from jax.experimental import pallas as pl
from jax.experimental.pallas import tpu as pltpu
from jax.experimental.pallas import tpu_sc as plsc

_B, _N, _C = 16, 65536, 10
_BN = _B * _N
_MIN_ROI = (0.0, -51.2, -5.0)
_GRID = 0.4

_NC, _NS, _L = 2, 16, 16
_NW = _NC * _NS
_NG = _BN // 128
_GW = _NG // _NW
_GC = 32
_NCH = _GW // _GC
_NBUF = 2


def _ceil_idx(v, min_v):
    q = (v - jnp.float32(min_v)) / jnp.float32(_GRID)
    t = q.astype(jnp.int32)
    tf = t.astype(jnp.float32)
    return jnp.where(q > tf, t + 1, t)


def _sc_body(in_hbm, feat_hbm, idx_hbm, in_v, out_v, sem_in, sem_idx,
             sem_feat):
    wid = lax.axis_index("s") * _NC + lax.axis_index("c")
    b = wid // 2
    g0 = wid * _GW
    nh0 = (wid % 2) * _GW
    b_hi = b // 8
    b_lo = b % 8
    x_min, y_min, z_min = _MIN_ROI

    vb = jnp.full((_L,), b, dtype=jnp.int32)

    def prefill(i, cy):
        for u in range(_NBUF):
            for k in range(128 // _L):
                out_v[u, i, 0, pl.ds(k * _L, _L)] = vb
        return cy

    lax.fori_loop(0, _GC, prefill, 0)

    def issue_in(ch, buf):
        tc = nh0 + ch * _GC
        return [
            pltpu.async_copy(
                in_hbm.at[c, b_hi, pl.ds(tc, _GC), b_lo, :],
                in_v.at[buf, c], sem_in)
            for c in range(_C)
        ]

    in_h = [None] * _NCH
    feat_h = [None] * _NCH
    idx_h = [None] * _NCH
    for ch in range(_NBUF):
        in_h[ch] = issue_in(ch, ch % _NBUF)

    for ch in range(_NCH):
        buf = ch % _NBUF
        for h in in_h[ch]:
            h.wait()
        g = g0 + ch * _GC
        feat_h[ch] = [
            pltpu.async_copy(
                in_v.at[buf, c],
                feat_hbm.at[c // 8, pl.ds(g, _GC), c % 8, :], sem_feat)
            for c in range(_C)
        ]
        if ch >= _NBUF:
            idx_h[ch - _NBUF].wait()

        def step(i, cy, buf=buf):
            for k in range(128 // _L):
                s = pl.ds(k * _L, _L)
                x = in_v[buf, 0, i, s]
                y = in_v[buf, 1, i, s]
                z = in_v[buf, 2, i, s]
                out_v[buf, i, 1, s] = _ceil_idx(z, z_min)
                out_v[buf, i, 2, s] = _ceil_idx(y, y_min)
                out_v[buf, i, 3, s] = _ceil_idx(x, x_min)
            return cy

        lax.fori_loop(0, _GC, step, 0)
        idx_h[ch] = pltpu.async_copy(
            out_v.at[buf], idx_hbm.at[pl.ds(g, _GC), :, :], sem_idx)
        if ch + _NBUF < _NCH:
            for h in feat_h[ch]:
                h.wait()
            in_h[ch + _NBUF] = issue_in(ch + _NBUF, buf)

    for ch in range(_NCH - _NBUF, _NCH):
        for h in feat_h[ch]:
            h.wait()
        idx_h[ch].wait()


_sc_call = pl.kernel(
    _sc_body,
    out_type=(
        jax.ShapeDtypeStruct((2, _NG, 8, 128), jnp.float32),
        jax.ShapeDtypeStruct((_NG, 4, 128), jnp.int32),
    ),
    mesh=plsc.VectorSubcoreMesh(core_axis_name="c", subcore_axis_name="s"),
    compiler_params=pltpu.CompilerParams(needs_layout_passes=False),
    scratch_types=[
        pltpu.VMEM((_NBUF, _C, _GC, 128), jnp.float32),
        pltpu.VMEM((_NBUF, _GC, 4, 128), jnp.int32),
        pltpu.SemaphoreType.DMA,
        pltpu.SemaphoreType.DMA,
        pltpu.SemaphoreType.DMA,
    ],
)


def kernel(rdr_sparse_cube):
    xv = (rdr_sparse_cube.transpose(2, 0, 1)
          .reshape(_C, 2, 8, 512, 128)
          .transpose(0, 1, 3, 2, 4))
    f4, i3 = _sc_call(xv)
    feat = (f4.transpose(0, 2, 1, 3).reshape(16, _BN).T)[:, :_C]
    idx = i3.transpose(0, 2, 1).reshape(_BN, 4)
    return feat, idx
